# baseline (speedup 1.0000x reference)
import functools

import jax
import jax.numpy as jnp
from jax.experimental import pallas as pl
from jax.experimental.pallas import tpu as pltpu

B, T, H = 2, 2048, 768
HEADS = 4
LAYERS = 3
WINDOW = 2
NEG = 0.2
EPS = 1e-5
NEG_BIG = -1e30


def _shift_rows(arr, k):
    if k == 0:
        return arr
    n = arr.shape[0]
    z = jnp.zeros((abs(k),) + arr.shape[1:], arr.dtype)
    if k > 0:
        return jnp.concatenate([arr[k:], z], axis=0)
    return jnp.concatenate([z, arr[: n + k]], axis=0)


def _body(x_ref, w_ref, asrc_ref, adst_ref, bias_ref, gamma_ref, beta_ref,
          out_ref, nf_ref, nf16_ref, acc_ref, sem):
    b = pl.program_id(0)
    l = pl.program_id(1)
    hd = pl.program_id(2)

    @pl.when((l == 0) & (hd == 0))
    def _():
        pltpu.make_async_copy(x_ref.at[b], nf_ref, sem).start()
        pltpu.make_async_copy(x_ref.at[b], nf_ref, sem).wait()

    @pl.when(hd == 0)
    def _():
        nf16_ref[...] = nf_ref[...].astype(jnp.bfloat16)

    xb16 = nf16_ref[...]
    w_pair = w_ref[0]

    t_idx = jax.lax.broadcasted_iota(jnp.int32, (T, 1), 0)
    offs = list(range(-WINDOW, WINDOW + 1))
    xs = [_shift_rows(xb16, k) for k in offs]

    out_h = None
    for j in range(2):
        w_h = w_pair[:, j * H:(j + 1) * H]
        att2 = jnp.concatenate(
            [asrc_ref[0, j], adst_ref[0, j]], axis=0).T
        vsd = jnp.dot(w_h, att2.astype(jnp.bfloat16),
                      preferred_element_type=jnp.float32)
        lg = jnp.dot(xb16, vsd.astype(jnp.bfloat16),
                     preferred_element_type=jnp.float32)
        a_s = lg[:, 0:1]
        a_d = lg[:, 1:2]

        exs = []
        for k in offs:
            valid = (t_idx + k >= 0) & (t_idx + k < T)
            e = _shift_rows(a_s, k) + a_d
            e = jnp.where(e > 0, e, NEG * e)
            exs.append(jnp.exp(jnp.where(valid, e, NEG_BIG)))
        den = functools.reduce(jnp.add, exs) + 1e-16
        inv_den = (1.0 / HEADS) / den

        als = [(ex * inv_den).astype(jnp.bfloat16) for ex in exs]
        cs = [al * xk for al, xk in zip(als, xs)]
        xmix = ((cs[0] + cs[1]) + (cs[2] + cs[3])) + cs[4]
        oh = jnp.dot(xmix, w_h, preferred_element_type=jnp.float32)
        out_h = oh if out_h is None else out_h + oh
    out_h = out_h.astype(jnp.bfloat16)

    @pl.when(hd == 0)
    def _():
        acc_ref[...] = out_h

    @pl.when(hd != 0)
    def _():
        acc_ref[...] = acc_ref[...] + out_h

    @pl.when(hd == HEADS // 2 - 1)
    def _():
        z = acc_ref[...].astype(jnp.float32) + bias_ref[0] + nf_ref[...]
        mu = jnp.mean(z, axis=1, keepdims=True)
        var = jnp.mean((z - mu) ** 2, axis=1, keepdims=True)
        y = (z - mu) * jax.lax.rsqrt(var + EPS) * gamma_ref[0] + beta_ref[0]
        nf_ref[...] = y

        @pl.when(l == LAYERS - 1)
        def _():
            pltpu.make_async_copy(nf_ref, out_ref.at[b], sem).start()
            pltpu.make_async_copy(nf_ref, out_ref.at[b], sem).wait()


def kernel(x, W, att_src, att_dst, bias, gamma, beta):
    W16 = W.astype(jnp.bfloat16)
    asrc = att_src.reshape(LAYERS, HEADS, 1, H)
    adst = att_dst.reshape(LAYERS, HEADS, 1, H)
    b3 = bias.reshape(LAYERS, 1, H)
    g3 = gamma.reshape(LAYERS, 1, H)
    be3 = beta.reshape(LAYERS, 1, H)

    return pl.pallas_call(
        _body,
        grid=(B, LAYERS, HEADS // 2),
        in_specs=[
            pl.BlockSpec(memory_space=pltpu.MemorySpace.HBM),
            pl.BlockSpec((1, H, 2 * H), lambda b, l, h: (l, 0, h)),
            pl.BlockSpec((1, 2, 1, H), lambda b, l, h: (l, h, 0, 0)),
            pl.BlockSpec((1, 2, 1, H), lambda b, l, h: (l, h, 0, 0)),
            pl.BlockSpec((1, 1, H), lambda b, l, h: (l, 0, 0)),
            pl.BlockSpec((1, 1, H), lambda b, l, h: (l, 0, 0)),
            pl.BlockSpec((1, 1, H), lambda b, l, h: (l, 0, 0)),
        ],
        out_specs=pl.BlockSpec(memory_space=pltpu.MemorySpace.HBM),
        out_shape=jax.ShapeDtypeStruct((B, T, H), jnp.float32),
        scratch_shapes=[pltpu.VMEM((T, H), jnp.float32),
                        pltpu.VMEM((T, H), jnp.bfloat16),
                        pltpu.VMEM((T, H), jnp.bfloat16),
                        pltpu.SemaphoreType.DMA],
        compiler_params=pltpu.CompilerParams(
            dimension_semantics=("arbitrary", "arbitrary", "arbitrary")),
    )(x, W16, asrc, adst, b3, g3, be3)


# speedup vs baseline: 1.0296x; 1.0296x over previous
import functools

import jax
import jax.numpy as jnp
from jax.experimental import pallas as pl
from jax.experimental.pallas import tpu as pltpu

B, T, H = 2, 2048, 768
HEADS = 4
LAYERS = 3
WINDOW = 2
NEG = 0.2
EPS = 1e-5
NEG_BIG = -1e30


def _shift_rows(arr, k):
    if k == 0:
        return arr
    n = arr.shape[0]
    z = jnp.zeros((abs(k),) + arr.shape[1:], arr.dtype)
    if k > 0:
        return jnp.concatenate([arr[k:], z], axis=0)
    return jnp.concatenate([z, arr[: n + k]], axis=0)


def _body(x_ref, w_ref, vsd_ref, bias_ref, gamma_ref, beta_ref,
          out_ref, nf_ref, nf16_ref, acc_ref, sem):
    b = pl.program_id(0)
    l = pl.program_id(1)
    hd = pl.program_id(2)

    @pl.when((l == 0) & (hd == 0))
    def _():
        pltpu.make_async_copy(x_ref.at[b], nf_ref, sem).start()
        pltpu.make_async_copy(x_ref.at[b], nf_ref, sem).wait()

    @pl.when(hd == 0)
    def _():
        nf16_ref[...] = nf_ref[...].astype(jnp.bfloat16)

    xb16 = nf16_ref[...]
    w_pair = w_ref[0]
    h2 = jnp.dot(xb16, w_pair,
                 preferred_element_type=jnp.float32).astype(jnp.bfloat16)
    lg4 = jnp.dot(xb16, vsd_ref[0, 0],
                  preferred_element_type=jnp.float32)

    t_idx = jax.lax.broadcasted_iota(jnp.int32, (T, 1), 0)
    offs = list(range(-WINDOW, WINDOW + 1))

    out_h = None
    for j in range(2):
        h16 = h2[:, j * H:(j + 1) * H]
        a_s = lg4[:, 2 * j:2 * j + 1]
        a_d = lg4[:, 2 * j + 1:2 * j + 2]

        exs = []
        for k in offs:
            valid = (t_idx + k >= 0) & (t_idx + k < T)
            e = _shift_rows(a_s, k) + a_d
            e = jnp.maximum(e, NEG * e)
            exs.append(jnp.exp(jnp.where(valid, e, NEG_BIG)))
        den = functools.reduce(jnp.add, exs) + 1e-16
        inv_den = (1.0 / HEADS) / den

        als = [(ex * inv_den).astype(jnp.bfloat16) for ex in exs]
        cs = [al * _shift_rows(h16, k) for k, al in zip(offs, als)]
        oh = ((cs[0] + cs[1]) + (cs[2] + cs[3])) + cs[4]
        out_h = oh if out_h is None else out_h + oh

    @pl.when(hd == 0)
    def _():
        acc_ref[...] = out_h

    @pl.when(hd != 0)
    def _():
        acc_ref[...] = acc_ref[...] + out_h

    @pl.when(hd == HEADS // 2 - 1)
    def _():
        z = acc_ref[...].astype(jnp.float32) + bias_ref[0] + nf_ref[...]
        mu = jnp.mean(z, axis=1, keepdims=True)
        var = jnp.mean((z - mu) ** 2, axis=1, keepdims=True)
        y = (z - mu) * jax.lax.rsqrt(var + EPS) * gamma_ref[0] + beta_ref[0]
        nf_ref[...] = y

        @pl.when(l == LAYERS - 1)
        def _():
            pltpu.make_async_copy(nf_ref, out_ref.at[b], sem).start()
            pltpu.make_async_copy(nf_ref, out_ref.at[b], sem).wait()


def kernel(x, W, att_src, att_dst, bias, gamma, beta):
    W16 = W.astype(jnp.bfloat16)
    W4 = W.reshape(LAYERS, H, HEADS, H)
    vs = jnp.einsum('lchd,lhd->lch', W4, att_src)
    vd = jnp.einsum('lchd,lhd->lch', W4, att_dst)
    vsd = jnp.stack([vs, vd], axis=-1)
    vsd = vsd.reshape(LAYERS, H, 2, 4).transpose(0, 2, 1, 3)
    vsd = vsd.astype(jnp.bfloat16)
    b3 = bias.reshape(LAYERS, 1, H)
    g3 = gamma.reshape(LAYERS, 1, H)
    be3 = beta.reshape(LAYERS, 1, H)

    return pl.pallas_call(
        _body,
        grid=(B, LAYERS, HEADS // 2),
        in_specs=[
            pl.BlockSpec(memory_space=pltpu.MemorySpace.HBM),
            pl.BlockSpec((1, H, 2 * H), lambda b, l, h: (l, 0, h)),
            pl.BlockSpec((1, 1, H, 4), lambda b, l, h: (l, h, 0, 0)),
            pl.BlockSpec((1, 1, H), lambda b, l, h: (l, 0, 0)),
            pl.BlockSpec((1, 1, H), lambda b, l, h: (l, 0, 0)),
            pl.BlockSpec((1, 1, H), lambda b, l, h: (l, 0, 0)),
        ],
        out_specs=pl.BlockSpec(memory_space=pltpu.MemorySpace.HBM),
        out_shape=jax.ShapeDtypeStruct((B, T, H), jnp.float32),
        scratch_shapes=[pltpu.VMEM((T, H), jnp.float32),
                        pltpu.VMEM((T, H), jnp.bfloat16),
                        pltpu.VMEM((T, H), jnp.bfloat16),
                        pltpu.SemaphoreType.DMA],
        compiler_params=pltpu.CompilerParams(
            dimension_semantics=("arbitrary", "arbitrary", "arbitrary")),
    )(x, W16, vsd, b3, g3, be3)


# speedup vs baseline: 1.0711x; 1.0403x over previous
import functools

import jax
import jax.numpy as jnp
from jax.experimental import pallas as pl
from jax.experimental.pallas import tpu as pltpu

B, T, H = 2, 2048, 768
HEADS = 4
LAYERS = 3
WINDOW = 2
NEG = 0.2
EPS = 1e-5
NEG_BIG = -1e30


def _shift_rows(arr, k):
    if k == 0:
        return arr
    n = arr.shape[0]
    z = jnp.zeros((abs(k),) + arr.shape[1:], arr.dtype)
    if k > 0:
        return jnp.concatenate([arr[k:], z], axis=0)
    return jnp.concatenate([z, arr[: n + k]], axis=0)


def _body(x_ref, w_ref, asrc_ref, adst_ref, bias_ref, gamma_ref, beta_ref,
          out_ref, nf_ref, nf16_ref, acc_ref, sem):
    b = pl.program_id(0)
    l = pl.program_id(1)
    hd = pl.program_id(2)

    @pl.when((l == 0) & (hd == 0))
    def _():
        pltpu.make_async_copy(x_ref.at[b], nf_ref, sem).start()
        pltpu.make_async_copy(x_ref.at[b], nf_ref, sem).wait()

    @pl.when(hd == 0)
    def _():
        nf16_ref[...] = nf_ref[...].astype(jnp.bfloat16)

    xb16 = nf16_ref[...]
    w_pair = w_ref[0]
    h2 = jnp.dot(xb16, w_pair,
                 preferred_element_type=jnp.float32).astype(jnp.bfloat16)

    t_idx = jax.lax.broadcasted_iota(jnp.int32, (T, 1), 0)
    offs = list(range(-WINDOW, WINDOW + 1))

    out_h = None
    for j in range(2):
        h16 = h2[:, j * H:(j + 1) * H]
        w_h = w_pair[:, j * H:(j + 1) * H]
        att2 = jnp.concatenate(
            [asrc_ref[0, j], adst_ref[0, j]], axis=0).T
        vsd = jnp.dot(w_h, att2.astype(jnp.bfloat16),
                      preferred_element_type=jnp.float32)
        lg = jnp.dot(xb16, vsd.astype(jnp.bfloat16),
                     preferred_element_type=jnp.float32)
        a_s = lg[:, 0:1]
        a_d = lg[:, 1:2]

        exs = []
        for k in offs:
            valid = (t_idx + k >= 0) & (t_idx + k < T)
            e = _shift_rows(a_s, k) + a_d
            e = jnp.maximum(e, NEG * e)
            exs.append(jnp.exp(jnp.where(valid, e, NEG_BIG)))
        den = functools.reduce(jnp.add, exs) + 1e-16
        inv_den = (1.0 / HEADS) / den

        als = [(ex * inv_den).astype(jnp.bfloat16) for ex in exs]
        cs = [al * _shift_rows(h16, k) for k, al in zip(offs, als)]
        oh = ((cs[0] + cs[1]) + (cs[2] + cs[3])) + cs[4]
        out_h = oh if out_h is None else out_h + oh

    @pl.when(hd == 0)
    def _():
        acc_ref[...] = out_h

    @pl.when(hd != 0)
    def _():
        acc_ref[...] = acc_ref[...] + out_h

    @pl.when(hd == HEADS // 2 - 1)
    def _():
        z = acc_ref[...].astype(jnp.float32) + bias_ref[0] + nf_ref[...]
        mu = jnp.mean(z, axis=1, keepdims=True)
        var = jnp.mean((z - mu) ** 2, axis=1, keepdims=True)
        y = (z - mu) * jax.lax.rsqrt(var + EPS) * gamma_ref[0] + beta_ref[0]
        nf_ref[...] = y

        @pl.when(l == LAYERS - 1)
        def _():
            pltpu.make_async_copy(nf_ref, out_ref.at[b], sem).start()
            pltpu.make_async_copy(nf_ref, out_ref.at[b], sem).wait()


def kernel(x, W, att_src, att_dst, bias, gamma, beta):
    W16 = W.astype(jnp.bfloat16)
    asrc = att_src.reshape(LAYERS, HEADS, 1, H)
    adst = att_dst.reshape(LAYERS, HEADS, 1, H)
    b3 = bias.reshape(LAYERS, 1, H)
    g3 = gamma.reshape(LAYERS, 1, H)
    be3 = beta.reshape(LAYERS, 1, H)

    return pl.pallas_call(
        _body,
        grid=(B, LAYERS, HEADS // 2),
        in_specs=[
            pl.BlockSpec(memory_space=pltpu.MemorySpace.HBM),
            pl.BlockSpec((1, H, 2 * H), lambda b, l, h: (l, 0, h)),
            pl.BlockSpec((1, 2, 1, H), lambda b, l, h: (l, h, 0, 0)),
            pl.BlockSpec((1, 2, 1, H), lambda b, l, h: (l, h, 0, 0)),
            pl.BlockSpec((1, 1, H), lambda b, l, h: (l, 0, 0)),
            pl.BlockSpec((1, 1, H), lambda b, l, h: (l, 0, 0)),
            pl.BlockSpec((1, 1, H), lambda b, l, h: (l, 0, 0)),
        ],
        out_specs=pl.BlockSpec(memory_space=pltpu.MemorySpace.HBM),
        out_shape=jax.ShapeDtypeStruct((B, T, H), jnp.float32),
        scratch_shapes=[pltpu.VMEM((T, H), jnp.float32),
                        pltpu.VMEM((T, H), jnp.bfloat16),
                        pltpu.VMEM((T, H), jnp.bfloat16),
                        pltpu.SemaphoreType.DMA],
        compiler_params=pltpu.CompilerParams(
            dimension_semantics=("arbitrary", "arbitrary", "arbitrary")),
    )(x, W16, asrc, adst, b3, g3, be3)
